# baseline (speedup 1.0000x reference)
import functools

import jax
import jax.numpy as jnp
from jax import lax
from jax.experimental import pallas as pl
from jax.experimental.pallas import tpu as pltpu
from jax.experimental.pallas import tpu_sc as plsc

_N = 10000
_E = 160000
_B = 8

_NC = 2
_NS = 16
_K = 128
_EPT = 5120
_EPAD = _NC * _NS * _EPT
_NCHUNK = _EPT // _K
_NP = 10240
_RPS = _NP // _NS


def _make_agg(C):
  mesh = plsc.VectorSubcoreMesh(core_axis_name="c", subcore_axis_name="s")

  @functools.partial(
      pl.kernel,
      mesh=mesh,
      out_type=jax.ShapeDtypeStruct((_NC, _NP, C), jnp.float32),
      scratch_types=[
          pltpu.VMEM((_K,), jnp.int32),
          pltpu.VMEM((_K,), jnp.int32),
          pltpu.VMEM((_K, 16), jnp.float32),
          pltpu.VMEM((_K, C), jnp.float32),
          pltpu.VMEM_SHARED((_NP, C), jnp.float32),
          pltpu.SemaphoreType.DMA,
      ],
  )
  def agg(vec_hbm, cols_hbm, rows_hbm, wrep_hbm, zeros_hbm, out_hbm,
          idxc, idxr, wv, rowsv, acc, sem):
    cid = lax.axis_index("c")
    sid = lax.axis_index("s")
    pltpu.sync_copy(zeros_hbm.at[pl.ds(sid * _RPS, _RPS)],
                    acc.at[pl.ds(sid * _RPS, _RPS)])
    plsc.subcore_barrier()

    base0 = (cid * _NS + sid) * _EPT

    def chunk_body(ci, _):
      b = pl.multiple_of(base0 + ci * _K, 8)
      pltpu.sync_copy(cols_hbm.at[pl.ds(b, _K)], idxc)
      pltpu.sync_copy(rows_hbm.at[pl.ds(b, _K)], idxr)
      pltpu.sync_copy(wrep_hbm.at[pl.ds(b, _K)], wv)
      pltpu.async_copy(vec_hbm.at[idxc], rowsv, sem).wait()

      def ebody(e, carry):
        for j in range(C // 16):
          rowsv[e, pl.ds(16 * j, 16)] = rowsv[e, pl.ds(16 * j, 16)] * wv[e, :]
        return carry

      lax.fori_loop(0, _K, ebody, 0)
      pltpu.sync_copy(rowsv, acc.at[idxr], add=True)
      return _

    lax.fori_loop(0, _NCHUNK, chunk_body, 0)
    plsc.subcore_barrier()
    pltpu.sync_copy(acc.at[pl.ds(sid * _RPS, _RPS)],
                    out_hbm.at[cid, pl.ds(sid * _RPS, _RPS)])

  return agg


_BN = 400


def _stage(aggs, bias, skip, k, relu_in, emit_h):
  nch = len(aggs)
  gin = bias.shape[1]
  gout = k.shape[2]
  grid = _N // _BN

  def body(*refs):
    i = 0
    parts = []
    for _c in range(nch):
      a = refs[i]; i += 1
      parts.append(a[0] + a[1])
    hb = parts[0] if nch == 1 else jnp.concatenate(parts, axis=-1)
    bref = refs[i]; i += 1
    h = (hb + bref[...]).reshape(_BN, _B, gin)
    if skip is not None:
      h = h + refs[i][...]; i += 1
    t = jnp.maximum(h, 0.0) if relu_in else h
    kref = refs[i][...]; i += 1
    acc = t[:, :, 0:1] * kref[:, 0:gout][:, None, :]
    for f in range(1, gin):
      acc = acc + t[:, :, f:f + 1] * kref[:, f * gout:(f + 1) * gout][:, None, :]
    refs[i][...] = acc; i += 1
    if emit_h:
      refs[i][...] = h

  in_specs = [pl.BlockSpec((2, _BN, a.shape[2]), lambda i: (0, i, 0))
              for a in aggs]
  in_specs.append(pl.BlockSpec((_BN, _B * gin), lambda i: (i, 0)))
  args = list(aggs) + [jnp.tile(bias, (1, _B))]
  if skip is not None:
    in_specs.append(pl.BlockSpec((_BN, _B, gin), lambda i: (i, 0, 0)))
    args.append(skip)
  in_specs.append(pl.BlockSpec((_BN, gin * gout), lambda i: (i, 0)))
  args.append(k.reshape(_N, gin * gout))

  out_shapes = [jax.ShapeDtypeStruct((_N, _B, gout), jnp.float32)]
  out_specs = [pl.BlockSpec((_BN, _B, gout), lambda i: (i, 0, 0))]
  if emit_h:
    out_shapes.append(jax.ShapeDtypeStruct((_N, _B, gin), jnp.float32))
    out_specs.append(pl.BlockSpec((_BN, _B, gin), lambda i: (i, 0, 0)))

  res = pl.pallas_call(
      body,
      grid=(grid,),
      in_specs=in_specs,
      out_specs=out_specs,
      out_shape=out_shapes,
  )(*args)
  return res if emit_h else (res[0], None)


def _chunks(vec3d):
  g = vec3d.shape[2]
  c = _B * g
  v2 = vec3d.reshape(_N, c)
  if c <= 128:
    return [v2]
  return [v2[:, :128], v2[:, 128:]]


def kernel(x, edge_index, edge_weight, k0, b0, k1a, b1a, k1b, b1b,
           k2a, b2a, k2b, b2b, kout, bout):
  rows = edge_index[0]
  cols = edge_index[1]
  pad = _EPAD - _E
  rows_p = jnp.concatenate([rows, jnp.zeros((pad,), jnp.int32)])
  cols_p = jnp.concatenate([cols, jnp.zeros((pad,), jnp.int32)])
  w_p = jnp.concatenate([edge_weight, jnp.zeros((pad,), jnp.float32)])
  wrep = jnp.broadcast_to(w_p[:, None], (_EPAD, 16)).astype(jnp.float32)
  wrep = jnp.asarray(wrep)

  agg128 = _make_agg(128)
  z128 = jnp.zeros((_NP, 128), jnp.float32)

  def run_agg(vec3d):
    return [agg128(ch, cols_p, rows_p, wrep, z128) for ch in _chunks(vec3d)]

  x_t = jnp.transpose(x, (1, 0, 2))
  x_p = jnp.pad(x_t, ((0, 0), (0, 0), (0, 13)))
  x_stack = jnp.stack([x_p.reshape(_N, _B * 16),
                       jnp.zeros((_N, _B * 16), jnp.float32)])
  k0_p = jnp.pad(k0, ((0, 0), (0, 13), (0, 0)))
  zb3 = jnp.zeros((_N, 16), jnp.float32)

  kout_p = jnp.pad(kout, ((0, 0), (0, 0), (0, 13)))
  bout_p = jnp.pad(bout, ((0, 0), (0, 13)))

  vec, _ = _stage([x_stack], zb3, None, k0_p, relu_in=False, emit_h=False)
  a = run_agg(vec)
  vec, skip1 = _stage(a, b0, None, k1a, relu_in=True, emit_h=True)
  a = run_agg(vec)
  vec, _ = _stage(a, b1a, None, k1b, relu_in=True, emit_h=False)
  a = run_agg(vec)
  vec, skip2 = _stage(a, b1b, skip1, k2a, relu_in=True, emit_h=True)
  a = run_agg(vec)
  vec, _ = _stage(a, b2a, None, k2b, relu_in=True, emit_h=False)
  a = run_agg(vec)
  vec, _ = _stage(a, b2b, skip2, kout_p, relu_in=True, emit_h=False)
  a = run_agg(vec)
  kdummy = jnp.zeros((_N, 16, 8), jnp.float32)
  _, fin = _stage(a, bout_p, None, kdummy, relu_in=False, emit_h=True)
  return jnp.transpose(fin[:, :, :3], (1, 0, 2))


# speedup vs baseline: 1.1262x; 1.1262x over previous
import functools

import jax
import jax.numpy as jnp
from jax import lax
from jax.experimental import pallas as pl
from jax.experimental.pallas import tpu as pltpu
from jax.experimental.pallas import tpu_sc as plsc

_N = 10000
_E = 160000
_B = 8

_NC = 2
_NS = 16
_K = 64
_EPT = 5120
_EPAD = _NC * _NS * _EPT
_NCHUNK = _EPT // _K
_NP = 10240
_RPS = _NP // _NS


def _make_agg(C):
  mesh = plsc.VectorSubcoreMesh(core_axis_name="c", subcore_axis_name="s")

  @functools.partial(
      pl.kernel,
      mesh=mesh,
      out_type=jax.ShapeDtypeStruct((_NC, _NP, C), jnp.float32),
      scratch_types=[
          pltpu.VMEM((2, _K), jnp.int32),
          pltpu.VMEM((2, _K), jnp.int32),
          pltpu.VMEM((2, _K, 16), jnp.float32),
          pltpu.VMEM((2, _K, C), jnp.float32),
          pltpu.VMEM_SHARED((_NP, C), jnp.float32),
          pltpu.SemaphoreType.DMA,
          pltpu.SemaphoreType.DMA,
      ],
  )
  def agg(vec_hbm, cols_hbm, rows_hbm, wrep_hbm, zeros_hbm, out_hbm,
          idxc, idxr, wv, rowsv, acc, sem0, sem1):
    cid = lax.axis_index("c")
    sid = lax.axis_index("s")
    sems = (sem0, sem1)
    pltpu.sync_copy(zeros_hbm.at[pl.ds(sid * _RPS, _RPS)],
                    acc.at[pl.ds(sid * _RPS, _RPS)])
    plsc.subcore_barrier()

    base0 = (cid * _NS + sid) * _EPT

    def prefetch(ci, buf):
      b = pl.multiple_of(base0 + ci * _K, 8)
      pltpu.sync_copy(cols_hbm.at[pl.ds(b, _K)], idxc.at[buf])
      pltpu.sync_copy(rows_hbm.at[pl.ds(b, _K)], idxr.at[buf])
      pltpu.sync_copy(wrep_hbm.at[pl.ds(b, _K)], wv.at[buf])
      pltpu.async_copy(vec_hbm.at[idxc.at[buf]], rowsv.at[buf], sems[buf])

    prefetch(0, 0)
    prefetch(1, 1)

    def step(g, carry):
      for buf in (0, 1):
        ci = 2 * g + buf
        pltpu.make_async_copy(vec_hbm.at[idxc.at[buf]], rowsv.at[buf],
                              sems[buf]).wait()

        def ebody(e, c2):
          for j in range(C // 16):
            rowsv[buf, e, pl.ds(16 * j, 16)] = (
                rowsv[buf, e, pl.ds(16 * j, 16)] * wv[buf, e, :])
          return c2

        lax.fori_loop(0, _K, ebody, 0)
        pltpu.sync_copy(rowsv.at[buf], acc.at[idxr.at[buf]], add=True)

        @pl.when(ci + 2 < _NCHUNK)
        def _do_prefetch():
          prefetch(ci + 2, buf)
      return carry

    lax.fori_loop(0, _NCHUNK // 2, step, 0)
    plsc.subcore_barrier()
    pltpu.sync_copy(acc.at[pl.ds(sid * _RPS, _RPS)],
                    out_hbm.at[cid, pl.ds(sid * _RPS, _RPS)])

  return agg


_BN = 400


def _stage(aggs, bias, skip, k, relu_in, emit_h):
  nch = len(aggs)
  gin = bias.shape[1]
  gout = k.shape[2]
  grid = _N // _BN

  def body(*refs):
    i = 0
    parts = []
    for _c in range(nch):
      a = refs[i]; i += 1
      parts.append(a[0] + a[1])
    hb = parts[0] if nch == 1 else jnp.concatenate(parts, axis=-1)
    bref = refs[i]; i += 1
    h = (hb + bref[...]).reshape(_BN, _B, gin)
    if skip is not None:
      h = h + refs[i][...]; i += 1
    t = jnp.maximum(h, 0.0) if relu_in else h
    kref = refs[i][...]; i += 1
    acc = t[:, :, 0:1] * kref[:, 0:gout][:, None, :]
    for f in range(1, gin):
      acc = acc + t[:, :, f:f + 1] * kref[:, f * gout:(f + 1) * gout][:, None, :]
    refs[i][...] = acc; i += 1
    if emit_h:
      refs[i][...] = h

  in_specs = [pl.BlockSpec((2, _BN, a.shape[2]), lambda i: (0, i, 0))
              for a in aggs]
  in_specs.append(pl.BlockSpec((_BN, _B * gin), lambda i: (i, 0)))
  args = list(aggs) + [jnp.tile(bias, (1, _B))]
  if skip is not None:
    in_specs.append(pl.BlockSpec((_BN, _B, gin), lambda i: (i, 0, 0)))
    args.append(skip)
  in_specs.append(pl.BlockSpec((_BN, gin * gout), lambda i: (i, 0)))
  args.append(k.reshape(_N, gin * gout))

  out_shapes = [jax.ShapeDtypeStruct((_N, _B, gout), jnp.float32)]
  out_specs = [pl.BlockSpec((_BN, _B, gout), lambda i: (i, 0, 0))]
  if emit_h:
    out_shapes.append(jax.ShapeDtypeStruct((_N, _B, gin), jnp.float32))
    out_specs.append(pl.BlockSpec((_BN, _B, gin), lambda i: (i, 0, 0)))

  res = pl.pallas_call(
      body,
      grid=(grid,),
      in_specs=in_specs,
      out_specs=out_specs,
      out_shape=out_shapes,
  )(*args)
  return res if emit_h else (res[0], None)


def _chunks(vec3d):
  g = vec3d.shape[2]
  c = _B * g
  v2 = vec3d.reshape(_N, c)
  if c <= 128:
    return [v2]
  return [v2[:, :128], v2[:, 128:]]


def kernel(x, edge_index, edge_weight, k0, b0, k1a, b1a, k1b, b1b,
           k2a, b2a, k2b, b2b, kout, bout):
  rows = edge_index[0]
  cols = edge_index[1]
  pad = _EPAD - _E
  rows_p = jnp.concatenate([rows, jnp.zeros((pad,), jnp.int32)])
  cols_p = jnp.concatenate([cols, jnp.zeros((pad,), jnp.int32)])
  w_p = jnp.concatenate([edge_weight, jnp.zeros((pad,), jnp.float32)])
  wrep = jnp.broadcast_to(w_p[:, None], (_EPAD, 16)).astype(jnp.float32)
  wrep = jnp.asarray(wrep)

  agg128 = _make_agg(128)
  z128 = jnp.zeros((_NP, 128), jnp.float32)

  def run_agg(vec3d):
    return [agg128(ch, cols_p, rows_p, wrep, z128) for ch in _chunks(vec3d)]

  x_t = jnp.transpose(x, (1, 0, 2))
  x_p = jnp.pad(x_t, ((0, 0), (0, 0), (0, 13)))
  x_stack = jnp.stack([x_p.reshape(_N, _B * 16),
                       jnp.zeros((_N, _B * 16), jnp.float32)])
  k0_p = jnp.pad(k0, ((0, 0), (0, 13), (0, 0)))
  zb3 = jnp.zeros((_N, 16), jnp.float32)

  kout_p = jnp.pad(kout, ((0, 0), (0, 0), (0, 13)))
  bout_p = jnp.pad(bout, ((0, 0), (0, 13)))

  vec, _ = _stage([x_stack], zb3, None, k0_p, relu_in=False, emit_h=False)
  a = run_agg(vec)
  vec, skip1 = _stage(a, b0, None, k1a, relu_in=True, emit_h=True)
  a = run_agg(vec)
  vec, _ = _stage(a, b1a, None, k1b, relu_in=True, emit_h=False)
  a = run_agg(vec)
  vec, skip2 = _stage(a, b1b, skip1, k2a, relu_in=True, emit_h=True)
  a = run_agg(vec)
  vec, _ = _stage(a, b2a, None, k2b, relu_in=True, emit_h=False)
  a = run_agg(vec)
  vec, _ = _stage(a, b2b, skip2, kout_p, relu_in=True, emit_h=False)
  a = run_agg(vec)
  kdummy = jnp.zeros((_N, 16, 8), jnp.float32)
  _, fin = _stage(a, bout_p, None, kdummy, relu_in=False, emit_h=True)
  return jnp.transpose(fin[:, :, :3], (1, 0, 2))
